# baseline (speedup 1.0000x reference)
import functools

import jax
import jax.numpy as jnp
from jax import lax
from jax.experimental import pallas as pl
from jax.experimental.pallas import tpu as pltpu
from jax.experimental.pallas import tpu_sc as plsc

_ROWS = 16384
_COLS = 200
_NW = 32
_RPW = _ROWS // _NW


def _sc_body(x_hbm, y_hbm, buf, _):
    c = lax.axis_index("c")
    s = lax.axis_index("s")
    wid = s * 2 + c
    base = wid * _RPW
    pltpu.sync_copy(x_hbm.at[pl.ds(base, _RPW), :], buf)

    zeros = jnp.zeros((16,), jnp.int32)
    ones = jnp.ones((16,), jnp.int32)

    def fix(i, carry):
        rows16 = i * 16 + lax.iota(jnp.int32, 16)
        c0 = plsc.load_gather(buf, [rows16, zeros])
        c1 = plsc.load_gather(buf, [rows16, ones])
        plsc.store_scatter(buf, [rows16, zeros], c1)
        plsc.store_scatter(buf, [rows16, ones], c0)
        return carry

    lax.fori_loop(0, _RPW // 16, fix, 0)
    pltpu.sync_copy(buf, y_hbm.at[pl.ds(base, _RPW), :])


def _make_sc_swap():
    mesh = plsc.VectorSubcoreMesh(core_axis_name="c", subcore_axis_name="s")
    return pl.kernel(
        _sc_body,
        out_type=jax.ShapeDtypeStruct((_ROWS, _COLS), jnp.float32),
        mesh=mesh,
        compiler_params=pltpu.CompilerParams(use_tc_tiling_on_sc=False, needs_layout_passes=False),
        scratch_types=[
            pltpu.VMEM((_RPW, _COLS), jnp.float32),
            pltpu.SemaphoreType.DMA,
        ],
    )


_sc_swap = _make_sc_swap()


def kernel(x):
    y = _sc_swap(x)
    return (x, y, x, y, x)
